# baseline (speedup 1.0000x reference)
import jax
import jax.numpy as jnp
from jax.experimental import pallas as pl

N = 10000
NEG = 0.2


def _copy_kernel(x_ref, o_ref):
    o_ref[...] = x_ref[...]


def _segment_softmax(alpha, dst, num_nodes):
    amax = jax.ops.segment_max(alpha, dst, num_segments=num_nodes)
    amax = jnp.where(jnp.isfinite(amax), amax, 0.0)
    a = jnp.exp(alpha - amax[dst])
    denom = jax.ops.segment_sum(a, dst, num_segments=num_nodes)
    return a / (denom[dst] + 1e-16)


def _layer(h_in, W, al, ar, b, src, dst, num_nodes):
    h = h_in @ W
    xj = h[src]
    xi = h[dst]
    logits = jnp.sum(xi * xj, axis=-1)
    alpha = jnp.sum(xj * al, axis=-1) + jnp.sum(xi * ar, axis=-1)
    alpha = alpha * jax.nn.sigmoid(logits)
    alpha = jax.nn.leaky_relu(alpha, NEG)
    alpha = _segment_softmax(alpha, dst, num_nodes)
    out = jax.ops.segment_sum(xj * alpha[:, None], dst, num_segments=num_nodes)
    return out + b


def kernel(x, edge_index, params):
    src0 = edge_index[0]
    dst0 = edge_index[1]
    keep = src0 != dst0
    dst0 = jnp.where(keep, dst0, N)
    loop = jnp.arange(N, dtype=edge_index.dtype)
    src = jnp.concatenate([src0, loop])
    dst = jnp.concatenate([dst0, loop])

    h = x @ params['W0'] + params['b0']
    h = pl.pallas_call(
        _copy_kernel,
        out_shape=jax.ShapeDtypeStruct(h.shape, h.dtype),
    )(h)
    for i in range(1, 16):
        h = _layer(h, params['W%d' % i], params['al%d' % i],
                   params['ar%d' % i], params['b%d' % i], src, dst, x.shape[0])
        h = jax.nn.relu(h)
    return h @ params['W16'] + params['b16']


# speedup vs baseline: 2.9331x; 2.9331x over previous
import functools

import jax
import jax.numpy as jnp
from jax import lax
from jax.experimental import pallas as pl
from jax.experimental.pallas import tpu as pltpu
from jax.experimental.pallas import tpu_sc as plsc

N = 10000
E = 320000
HID = 32
NEG = 0.2
NP = 10240
NV = 320
NWK = 32
EMAX = 16512
NCH_MAX = EMAX // 128


def _tc_first(x_ref, w0_ref, b0_ref, w1_ref, al_ref, ar_ref,
              hw_o, s_o, t_o, af_o):
    h = jnp.dot(x_ref[...], w0_ref[...]) + b0_ref[...]
    hw = jnp.dot(h, w1_ref[...])
    s = jnp.dot(hw, al_ref[...])
    t = jnp.dot(hw, ar_ref[...])
    ss = jnp.sum(hw * hw, axis=1, keepdims=True)
    af = (s + t) * jax.nn.sigmoid(ss)
    af = jnp.where(af >= 0, af, NEG * af)
    hw_o[...] = hw
    s_o[...] = s
    t_o[...] = t
    af_o[...] = af


def _tc_mid(acc_ref, den_ref, bp_ref, w_ref, al_ref, ar_ref,
            hw_o, s_o, t_o, af_o):
    h = acc_ref[...] / (den_ref[...] + 1e-16) + bp_ref[...]
    h = jnp.maximum(h, 0.0)
    hw = jnp.dot(h, w_ref[...])
    s = jnp.dot(hw, al_ref[...])
    t = jnp.dot(hw, ar_ref[...])
    ss = jnp.sum(hw * hw, axis=1, keepdims=True)
    af = (s + t) * jax.nn.sigmoid(ss)
    af = jnp.where(af >= 0, af, NEG * af)
    hw_o[...] = hw
    s_o[...] = s
    t_o[...] = t
    af_o[...] = af


def _tc_out(acc_ref, den_ref, bp_ref, w16_ref, b16_ref, o_ref):
    h = acc_ref[...] / (den_ref[...] + 1e-16) + bp_ref[...]
    h = jnp.maximum(h, 0.0)
    o_ref[...] = jnp.dot(h, w16_ref[...]) + b16_ref[...]


_DENSE_OUT = (
    jax.ShapeDtypeStruct((NP, HID), jnp.float32),
    jax.ShapeDtypeStruct((NP, 1), jnp.float32),
    jax.ShapeDtypeStruct((NP, 1), jnp.float32),
    jax.ShapeDtypeStruct((NP, 1), jnp.float32),
)

_tc_first_call = pl.pallas_call(_tc_first, out_shape=_DENSE_OUT)
_tc_mid_call = pl.pallas_call(_tc_mid, out_shape=_DENSE_OUT)
_tc_out_call = pl.pallas_call(
    _tc_out, out_shape=jax.ShapeDtypeStruct((NP, 128), jnp.float32))


def _sc_edge(hw, s, t, af, srcp, dstp, offs, acc_o, den_o,
             hloc, sloc, tloc, aloc, olc, srcbuf, dstbuf, abuf,
             ebtab, dgtab, mtab, dentab, acctab, gidx, rowsA, rowsB,
             semA, semB):
    cid = lax.axis_index("c")
    sid = lax.axis_index("s")
    wid = sid * 2 + cid
    v0 = pl.multiple_of(wid * NV, NV)
    iota = lax.iota(jnp.int32, 16)

    pltpu.sync_copy(offs.at[pl.ds(v0, 344)], olc.at[pl.ds(0, 344)])
    pltpu.sync_copy(hw.at[pl.ds(v0, NV)], hloc.at[pl.ds(0, NV)])
    pltpu.sync_copy(s, sloc)
    pltpu.sync_copy(t.at[pl.ds(v0, NV)], tloc.at[pl.ds(0, NV)])
    pltpu.sync_copy(af.at[pl.ds(v0, NV)], aloc.at[pl.ds(0, NV)])

    e0 = jnp.min(olc[pl.ds(0, 16)])
    e1 = jnp.min(olc[pl.ds(NV, 16)])
    e0a = pl.multiple_of((e0 // 128) * 128, 128)
    nch = jnp.minimum((e1 - e0a + 127) // 128, NCH_MAX)

    pltpu.sync_copy(srcp.at[pl.ds(e0a, EMAX)], srcbuf)
    pltpu.sync_copy(dstp.at[pl.ds(e0a, EMAX)], dstbuf)

    @pl.when(nch > 0)
    def _():
        pltpu.async_copy(hw.at[srcbuf.at[pl.ds(0, 128)]], rowsA.at[0],
                         semA.at[0])

    def pa_body(i, carry):
        ring = lax.rem(i, 2)
        pltpu.make_async_copy(hw.at[srcbuf.at[pl.ds(i * 128, 128)]],
                              rowsA.at[ring], semA.at[ring]).wait()

        @pl.when(i + 1 < nch)
        def _():
            rn = lax.rem(i + 1, 2)
            pltpu.async_copy(hw.at[srcbuf.at[pl.ds((i + 1) * 128, 128)]],
                             rowsA.at[rn], semA.at[rn])

        ringv = jnp.full((16,), ring, jnp.int32)

        def pg(g, c2):
            base = i * 128 + g * 16
            srcv = srcbuf[pl.ds(base, 16)]
            dstv = dstbuf[pl.ds(base, 16)]
            ev = e0a + base + iota
            valid = (ev >= e0) & (ev < e1)
            dl = jnp.where(valid, dstv - v0, NV)
            sv = plsc.load_gather(sloc, [srcv])
            tv = plsc.load_gather(tloc, [dl])
            rowv = g * 16 + iota

            def pc(c, lg):
                cv = jnp.full((16,), c, jnp.int32)
                xj = plsc.load_gather(rowsA, [ringv, rowv, cv])
                xi = plsc.load_gather(hloc, [dl, cv])
                return lg + xj * xi

            logit = lax.fori_loop(0, HID, pc, jnp.zeros((16,), jnp.float32))
            sig = 1.0 / (1.0 + jnp.exp(-logit))
            a = (sv + tv) * sig
            a = jnp.where(a >= 0, a, NEG * a)
            a = jnp.where(valid, a, -1e30)
            abuf[pl.ds(base, 16)] = a
            return c2

        lax.fori_loop(0, 8, pg, 0)
        return carry

    lax.fori_loop(0, nch, pa_body, 0)

    def tb(j, km):
        ol = olc[pl.ds(j * 16, 16)]
        oh = olc[pl.ds(j * 16 + 1, 16)]
        dg = oh - ol
        ebtab[pl.ds(j * 16, 16)] = ol - e0a
        dgtab[pl.ds(j * 16, 16)] = dg
        mtab[pl.ds(j * 16, 16)] = aloc[pl.ds(j * 16, 16)]
        return jnp.maximum(km, dg)

    km = lax.fori_loop(0, 20, tb, jnp.zeros((16,), jnp.int32))
    kmax = jnp.minimum(jnp.max(km), EMAX)

    def m_body(k, carry):
        def mj(j, c2):
            eb = ebtab[pl.ds(j * 16, 16)]
            dg = dgtab[pl.ds(j * 16, 16)]
            msk = k < dg
            ei = jnp.minimum(jnp.where(msk, eb + k, EMAX - 1), EMAX - 1)
            av = plsc.load_gather(abuf, [ei])
            mv = mtab[pl.ds(j * 16, 16)]
            mtab[pl.ds(j * 16, 16)] = jnp.maximum(
                mv, jnp.where(msk, av, -1e30))
            return c2
        return lax.fori_loop(0, 20, mj, carry)

    lax.fori_loop(0, kmax, m_body, 0)

    def init_j(j, carry):
        a0 = jnp.exp(aloc[pl.ds(j * 16, 16)] - mtab[pl.ds(j * 16, 16)])
        dentab[pl.ds(j * 16, 16)] = a0
        rowv = j * 16 + iota

        def ic(c, c2):
            cv = jnp.full((16,), c, jnp.int32)
            xi = plsc.load_gather(hloc, [rowv, cv])
            plsc.store_scatter(acctab, [rowv, cv], a0 * xi)
            return c2

        lax.fori_loop(0, HID, ic, 0)
        return carry

    lax.fori_loop(0, 20, init_j, 0)

    def build(k, ring):
        def bj(j, c2):
            eb = ebtab[pl.ds(j * 16, 16)]
            dg = dgtab[pl.ds(j * 16, 16)]
            msk = k < dg
            ei = jnp.minimum(jnp.where(msk, eb + k, EMAX - 1), EMAX - 1)
            srcv = plsc.load_gather(srcbuf, [ei])
            gidx[pl.ds(ring * NV + j * 16, 16)] = jnp.where(msk, srcv, 0)
            return c2
        lax.fori_loop(0, 20, bj, 0)

    def issue(ring):
        for sub in range(4):
            pltpu.async_copy(
                hw.at[gidx.at[pl.ds(ring * NV + sub * 80, 80)]],
                rowsB.at[ring, pl.ds(sub * 80, 80)], semB.at[ring])

    def waitb(ring):
        for sub in range(4):
            pltpu.make_async_copy(
                hw.at[gidx.at[pl.ds(ring * NV + sub * 80, 80)]],
                rowsB.at[ring, pl.ds(sub * 80, 80)], semB.at[ring]).wait()

    @pl.when(kmax > 0)
    def _():
        build(0, 0)
        issue(0)

    def pb_body(k, carry):
        ring = lax.rem(k, 2)
        waitb(ring)

        @pl.when(k + 1 < kmax)
        def _():
            rn = lax.rem(k + 1, 2)
            build(k + 1, rn)
            issue(rn)

        ringv = jnp.full((16,), ring, jnp.int32)

        def cj(j, c2):
            eb = ebtab[pl.ds(j * 16, 16)]
            dg = dgtab[pl.ds(j * 16, 16)]
            msk = k < dg
            ei = jnp.minimum(jnp.where(msk, eb + k, EMAX - 1), EMAX - 1)
            av = plsc.load_gather(abuf, [ei])
            mv = mtab[pl.ds(j * 16, 16)]
            a = jnp.where(msk, jnp.exp(av - mv), 0.0)
            dentab[pl.ds(j * 16, 16)] = dentab[pl.ds(j * 16, 16)] + a
            rowv = j * 16 + iota

            def cc(c, c3):
                cv = jnp.full((16,), c, jnp.int32)
                xj = plsc.load_gather(rowsB, [ringv, rowv, cv])
                accv = plsc.load_gather(acctab, [rowv, cv])
                plsc.store_scatter(acctab, [rowv, cv], accv + a * xj)
                return c3

            lax.fori_loop(0, HID, cc, 0)
            return c2

        lax.fori_loop(0, 20, cj, 0)
        return carry

    lax.fori_loop(0, kmax, pb_body, 0)

    pltpu.sync_copy(acctab, acc_o.at[pl.ds(v0, NV)])
    pltpu.sync_copy(dentab.at[pl.ds(0, NV)], den_o.at[pl.ds(v0, NV)])


_sc_edge_call = pl.kernel(
    _sc_edge,
    out_type=(
        jax.ShapeDtypeStruct((NP, HID), jnp.float32),
        jax.ShapeDtypeStruct((NP,), jnp.float32),
    ),
    mesh=plsc.VectorSubcoreMesh(
        core_axis_name="c", subcore_axis_name="s",
        num_cores=2, num_subcores=16),
    compiler_params=pltpu.CompilerParams(
        needs_layout_passes=False, use_tc_tiling_on_sc=False),
    scratch_types=[
        pltpu.VMEM((NV + 8, HID), jnp.float32),
        pltpu.VMEM((NP,), jnp.float32),
        pltpu.VMEM((NV + 16, ), jnp.float32),
        pltpu.VMEM((NV + 16, ), jnp.float32),
        pltpu.VMEM((352,), jnp.int32),
        pltpu.VMEM((EMAX,), jnp.int32),
        pltpu.VMEM((EMAX,), jnp.int32),
        pltpu.VMEM((EMAX,), jnp.float32),
        pltpu.VMEM((336,), jnp.int32),
        pltpu.VMEM((336,), jnp.int32),
        pltpu.VMEM((336,), jnp.float32),
        pltpu.VMEM((336,), jnp.float32),
        pltpu.VMEM((NV, HID), jnp.float32),
        pltpu.VMEM((2 * NV,), jnp.int32),
        pltpu.VMEM((2, 128, HID), jnp.float32),
        pltpu.VMEM((2, NV, HID), jnp.float32),
        pltpu.SemaphoreType.DMA((2,)),
        pltpu.SemaphoreType.DMA((2,)),
    ],
)


def kernel(x, edge_index, params):
    src0 = edge_index[0]
    dst0 = edge_index[1]
    keep = src0 != dst0
    dsts = jnp.where(keep, dst0, N)
    dst_s, src_s = lax.sort([dsts, src0], num_keys=1)
    offs = jnp.searchsorted(
        dst_s, jnp.arange(NP + 24, dtype=jnp.int32)).astype(jnp.int32)
    zpad = jnp.zeros((EMAX,), jnp.int32)
    srcp = jnp.concatenate([src_s, zpad])
    dstp = jnp.concatenate([dst_s, zpad])

    x_pad = jnp.pad(x, ((0, NP - N), (0, 0)))
    b0 = params['b0'].reshape(1, HID)
    b16 = params['b16'].reshape(1, 128)

    hw, s, t, af = _tc_first_call(
        x_pad, params['W0'], b0, params['W1'],
        params['al1'].reshape(HID, 1), params['ar1'].reshape(HID, 1))
    acc, den = _sc_edge_call(
        hw, s.reshape(NP), t.reshape(NP), af.reshape(NP), srcp, dstp, offs)

    ws = jnp.stack([params['W%d' % i] for i in range(2, 16)])
    als = jnp.stack([params['al%d' % i].reshape(HID, 1)
                     for i in range(2, 16)])
    ars = jnp.stack([params['ar%d' % i].reshape(HID, 1)
                     for i in range(2, 16)])
    bps = jnp.stack([params['b%d' % i].reshape(1, HID)
                     for i in range(1, 15)])

    def step(carry, ps):
        acc_c, den_c = carry
        w, al, ar, bp = ps
        hw_c, s_c, t_c, af_c = _tc_mid_call(
            acc_c, den_c.reshape(NP, 1), bp, w, al, ar)
        acc_n, den_n = _sc_edge_call(
            hw_c, s_c.reshape(NP), t_c.reshape(NP), af_c.reshape(NP),
            srcp, dstp, offs)
        return (acc_n, den_n), None

    (acc, den), _ = lax.scan(step, (acc, den), (ws, als, ars, bps))

    out = _tc_out_call(acc, den.reshape(NP, 1),
                       params['b15'].reshape(1, HID),
                       params['W16'], b16)
    return out[:N]
